# baseline (speedup 1.0000x reference)
import dataclasses

import jax
import jax.numpy as jnp
from jax import lax
from jax.experimental import pallas as pl
from jax.experimental.pallas import tpu as pltpu
from jax.experimental.pallas import tpu_sc as plsc

N_NODES = 100000
D = 128
B = 8192
K = 128
L = 16
NC, NS = 2, 16
NW = NC * NS
BW = B // NW
CB = 64
NCHUNK = BW // CB
NJ = D // L


def _ds16(j):
    return pl.ds(j * L, L)


def _tree_sum(vals):
    vals = list(vals)
    while len(vals) > 1:
        nxt = [vals[2 * i] + vals[2 * i + 1] for i in range(len(vals) // 2)]
        if len(vals) % 2:
            nxt.append(vals[-1])
        vals = nxt
    return vals[0]


def _transpose_reduce(acc_ref, acc_base, iota):
    rows = iota + acc_base
    return _tree_sum(
        plsc.load_gather(acc_ref, [rows, jnp.full((L,), j, jnp.int32)])
        for j in range(L))


def _score_block(rows_ref, row_base, q_vecs, acc_ref, acc_base, iota):
    for r in range(L):
        acc_ref[acc_base + r, pl.ds(0, L)] = _tree_sum(
            rows_ref[row_base + r, _ds16(j)] * q_vecs[j] for j in range(NJ))
    return _transpose_reduce(acc_ref, acc_base, iota)


def _sc_body(embs, heads, rels, tails, hn, tn, wrel,
             pos_out, neg_out,
             h_idx, r_idx, t_idx, qh, qt, pos_v,
             hn_idx, tn_idx, na0, nb0,
             acc_ref, acc_ref2, out_buf,
             sa0, sb0):
    wid = lax.axis_index("s") * NC + lax.axis_index("c")
    base = wid * BW
    iota = lax.iota(jnp.int32, L)

    def ga(buf, sem, bb):
        return pltpu.make_async_copy(embs.at[hn_idx.at[bb]], buf, sem)

    def gb(buf, sem, bb):
        return pltpu.make_async_copy(embs.at[tn_idx.at[bb]], buf, sem)

    @pl.loop(0, NCHUNK)
    def _chunk(c):
        plsc.subcore_barrier()
        cb = base + c * CB
        pltpu.sync_copy(hn.at[pl.ds(cb, CB)], hn_idx)
        pltpu.sync_copy(tn.at[pl.ds(cb, CB)], tn_idx)
        pltpu.sync_copy(heads.at[pl.ds(cb, CB)], h_idx)
        pltpu.sync_copy(rels.at[pl.ds(cb, CB)], r_idx)
        pltpu.sync_copy(tails.at[pl.ds(cb, CB)], t_idx)

        pltpu.sync_copy(embs.at[h_idx], na0.at[pl.ds(0, CB)])
        pltpu.sync_copy(wrel.at[r_idx], na0.at[pl.ds(CB, CB)])
        pltpu.sync_copy(embs.at[t_idx], nb0.at[pl.ds(0, CB)])

        @pl.loop(0, CB // L)
        def _q(g):
            gbase = g * L
            for r in range(L):
                acc = None
                for j in range(NJ):
                    sv = na0[gbase + r, _ds16(j)]
                    rv = na0[CB + gbase + r, _ds16(j)]
                    ov = nb0[gbase + r, _ds16(j)]
                    qhv = rv * ov
                    qtv = sv * rv
                    qh[gbase + r, _ds16(j)] = qhv
                    qt[gbase + r, _ds16(j)] = qtv
                    p = sv * qhv
                    acc = p if acc is None else acc + p
                acc_ref[r, pl.ds(0, L)] = acc
            pos_v[pl.ds(gbase, L)] = _transpose_reduce(acc_ref, 0, iota)

        pltpu.sync_copy(pos_v, pos_out.at[pl.ds(cb, CB)])
        ga(na0, sa0, 0).start()

        @pl.loop(0, CB)
        def _b(bb):
            gb(nb0, sb0, bb).start()
            qhv = [qh[bb, _ds16(j)] for j in range(NJ)]
            qtv = [qt[bb, _ds16(j)] for j in range(NJ)]

            ga(na0, sa0, bb).wait()

            @plsc.parallel_loop(0, K // L, unroll=2)
            def _g1(g):
                rb = g * L
                out_buf[bb, pl.ds(rb, L)] = _score_block(
                    na0, rb, qhv, acc_ref, rb, iota)

            @pl.when(bb < CB - 1)
            def _():
                ga(na0, sa0, bb + 1).start()

            gb(nb0, sb0, bb).wait()

            @plsc.parallel_loop(0, K // L, unroll=2)
            def _g2(g):
                rb = g * L
                out_buf[bb, pl.ds(K + rb, L)] = _score_block(
                    nb0, rb, qtv, acc_ref2, rb, iota)

        pltpu.sync_copy(out_buf, neg_out.at[pl.ds(cb, CB)])


@jax.jit
def _link_pred_sc(embs, heads, rels, tails, hn, tn, wrel):
    mesh = plsc.VectorSubcoreMesh(core_axis_name="c", subcore_axis_name="s")
    cp = pltpu.CompilerParams()
    if "needs_layout_passes" in pltpu.CompilerParams.__dataclass_fields__:
        cp = dataclasses.replace(cp, needs_layout_passes=False)
    f = pl.kernel(
        _sc_body,
        out_type=(
            jax.ShapeDtypeStruct((B,), jnp.float32),
            jax.ShapeDtypeStruct((B, 2 * K), jnp.float32),
        ),
        mesh=mesh,
        scratch_types=[
            pltpu.VMEM((CB,), jnp.int32),
            pltpu.VMEM((CB,), jnp.int32),
            pltpu.VMEM((CB,), jnp.int32),
            pltpu.VMEM((CB, D), jnp.float32),
            pltpu.VMEM((CB, D), jnp.float32),
            pltpu.VMEM((CB,), jnp.float32),
            pltpu.VMEM((CB, K), jnp.int32),
            pltpu.VMEM((CB, K), jnp.int32),
            pltpu.VMEM((K, D), jnp.float32),
            pltpu.VMEM((K, D), jnp.float32),
            pltpu.VMEM((K, L + 1), jnp.float32),
            pltpu.VMEM((K, L + 1), jnp.float32),
            pltpu.VMEM((CB, 2 * K), jnp.float32),
            pltpu.SemaphoreType.DMA,
            pltpu.SemaphoreType.DMA,
        ],
        compiler_params=cp,
    )
    return f(embs, heads, rels, tails, hn, tn, wrel)


def kernel(embs, pos_samples, head_negative_sample, tail_negative_sample,
           w_relation):
    heads = pos_samples[:, 0]
    rels = pos_samples[:, 1]
    tails = pos_samples[:, 2]
    pos, neg = _link_pred_sc(embs, heads, rels, tails,
                             head_negative_sample, tail_negative_sample,
                             w_relation)
    return pos[:, None], neg


# speedup vs baseline: 1.0046x; 1.0046x over previous
import dataclasses

import jax
import jax.numpy as jnp
from jax import lax
from jax.experimental import pallas as pl
from jax.experimental.pallas import tpu as pltpu
from jax.experimental.pallas import tpu_sc as plsc

N_NODES = 100000
D = 128
B = 8192
K = 128
L = 16
NC, NS = 2, 16
NW = NC * NS
BW = B // NW
CB = 64
NCHUNK = BW // CB
NJ = D // L


def _ds16(j):
    return pl.ds(j * L, L)


def _tree_sum(vals):
    vals = list(vals)
    while len(vals) > 1:
        nxt = [vals[2 * i] + vals[2 * i + 1] for i in range(len(vals) // 2)]
        if len(vals) % 2:
            nxt.append(vals[-1])
        vals = nxt
    return vals[0]


def _transpose_reduce(acc_ref, acc_base, iota):
    rows = iota + acc_base
    return _tree_sum(
        plsc.load_gather(acc_ref, [rows, jnp.full((L,), j, jnp.int32)])
        for j in range(L))


def _score_block(rows_ref, row_base, q_vecs, acc_ref, acc_base, iota):
    for r in range(L):
        acc_ref[acc_base + r, pl.ds(0, L)] = _tree_sum(
            rows_ref[row_base + r, _ds16(j)] * q_vecs[j] for j in range(NJ))
    return _transpose_reduce(acc_ref, acc_base, iota)


def _sc_body(embs, heads, rels, tails, hn, tn, wrel,
             pos_out, neg_out,
             h_idx, r_idx, t_idx, qh, qt, pos_v,
             hn_idx, tn_idx, na0, nb0,
             acc_ref, acc_ref2, out_buf,
             sa0, sb0):
    wid = lax.axis_index("s") * NC + lax.axis_index("c")
    base = wid * BW
    iota = lax.iota(jnp.int32, L)

    def ga(buf, sem, bb):
        return pltpu.make_async_copy(embs.at[hn_idx.at[bb]], buf, sem)

    def gb(buf, sem, bb):
        return pltpu.make_async_copy(embs.at[tn_idx.at[bb]], buf, sem)

    @pl.loop(0, NCHUNK)
    def _chunk(c):
        cb = base + c * CB
        pltpu.sync_copy(hn.at[pl.ds(cb, CB)], hn_idx)
        pltpu.sync_copy(tn.at[pl.ds(cb, CB)], tn_idx)
        pltpu.sync_copy(heads.at[pl.ds(cb, CB)], h_idx)
        pltpu.sync_copy(rels.at[pl.ds(cb, CB)], r_idx)
        pltpu.sync_copy(tails.at[pl.ds(cb, CB)], t_idx)

        pltpu.sync_copy(embs.at[h_idx], na0.at[pl.ds(0, CB)])
        pltpu.sync_copy(wrel.at[r_idx], na0.at[pl.ds(CB, CB)])
        pltpu.sync_copy(embs.at[t_idx], nb0.at[pl.ds(0, CB)])

        @pl.loop(0, CB // L)
        def _q(g):
            gbase = g * L
            for r in range(L):
                acc = None
                for j in range(NJ):
                    sv = na0[gbase + r, _ds16(j)]
                    rv = na0[CB + gbase + r, _ds16(j)]
                    ov = nb0[gbase + r, _ds16(j)]
                    qhv = rv * ov
                    qtv = sv * rv
                    qh[gbase + r, _ds16(j)] = qhv
                    qt[gbase + r, _ds16(j)] = qtv
                    p = sv * qhv
                    acc = p if acc is None else acc + p
                acc_ref[r, pl.ds(0, L)] = acc
            pos_v[pl.ds(gbase, L)] = _transpose_reduce(acc_ref, 0, iota)

        pltpu.sync_copy(pos_v, pos_out.at[pl.ds(cb, CB)])
        ga(na0, sa0, 0).start()

        @pl.loop(0, CB)
        def _b(bb):
            gb(nb0, sb0, bb).start()
            qhv = [qh[bb, _ds16(j)] for j in range(NJ)]
            qtv = [qt[bb, _ds16(j)] for j in range(NJ)]

            ga(na0, sa0, bb).wait()

            @plsc.parallel_loop(0, K // L, unroll=2)
            def _g1(g):
                rb = g * L
                out_buf[bb, pl.ds(rb, L)] = _score_block(
                    na0, rb, qhv, acc_ref, rb, iota)

            @pl.when(bb < CB - 1)
            def _():
                ga(na0, sa0, bb + 1).start()

            gb(nb0, sb0, bb).wait()

            @plsc.parallel_loop(0, K // L, unroll=2)
            def _g2(g):
                rb = g * L
                out_buf[bb, pl.ds(K + rb, L)] = _score_block(
                    nb0, rb, qtv, acc_ref2, rb, iota)

        pltpu.sync_copy(out_buf, neg_out.at[pl.ds(cb, CB)])


@jax.jit
def _link_pred_sc(embs, heads, rels, tails, hn, tn, wrel):
    mesh = plsc.VectorSubcoreMesh(core_axis_name="c", subcore_axis_name="s")
    cp = pltpu.CompilerParams()
    if "needs_layout_passes" in pltpu.CompilerParams.__dataclass_fields__:
        cp = dataclasses.replace(cp, needs_layout_passes=False)
    f = pl.kernel(
        _sc_body,
        out_type=(
            jax.ShapeDtypeStruct((B,), jnp.float32),
            jax.ShapeDtypeStruct((B, 2 * K), jnp.float32),
        ),
        mesh=mesh,
        scratch_types=[
            pltpu.VMEM((CB,), jnp.int32),
            pltpu.VMEM((CB,), jnp.int32),
            pltpu.VMEM((CB,), jnp.int32),
            pltpu.VMEM((CB, D), jnp.float32),
            pltpu.VMEM((CB, D), jnp.float32),
            pltpu.VMEM((CB,), jnp.float32),
            pltpu.VMEM((CB, K), jnp.int32),
            pltpu.VMEM((CB, K), jnp.int32),
            pltpu.VMEM((K, D), jnp.float32),
            pltpu.VMEM((K, D), jnp.float32),
            pltpu.VMEM((K, L + 1), jnp.float32),
            pltpu.VMEM((K, L + 1), jnp.float32),
            pltpu.VMEM((CB, 2 * K), jnp.float32),
            pltpu.SemaphoreType.DMA,
            pltpu.SemaphoreType.DMA,
        ],
        compiler_params=cp,
    )
    return f(embs, heads, rels, tails, hn, tn, wrel)


def kernel(embs, pos_samples, head_negative_sample, tail_negative_sample,
           w_relation):
    heads = pos_samples[:, 0]
    rels = pos_samples[:, 1]
    tails = pos_samples[:, 2]
    pos, neg = _link_pred_sc(embs, heads, rels, tails,
                             head_negative_sample, tail_negative_sample,
                             w_relation)
    return pos[:, None], neg


# speedup vs baseline: 1.0207x; 1.0160x over previous
import dataclasses

import jax
import jax.numpy as jnp
from jax import lax
from jax.experimental import pallas as pl
from jax.experimental.pallas import tpu as pltpu
from jax.experimental.pallas import tpu_sc as plsc

N_NODES = 100000
D = 128
B = 8192
K = 128
L = 16
NC, NS = 2, 16
NW = NC * NS
BW = B // NW
CB = 64
NCHUNK = BW // CB
NJ = D // L


def _ds16(j):
    return pl.ds(j * L, L)


def _tree_sum(vals):
    vals = list(vals)
    while len(vals) > 1:
        nxt = [vals[2 * i] + vals[2 * i + 1] for i in range(len(vals) // 2)]
        if len(vals) % 2:
            nxt.append(vals[-1])
        vals = nxt
    return vals[0]


def _transpose_reduce(acc_ref, acc_base, iota):
    rows = iota + acc_base
    return _tree_sum(
        plsc.load_gather(acc_ref, [rows, jnp.full((L,), j, jnp.int32)])
        for j in range(L))


def _score_block(rows_ref, row_base, q_vecs, acc_ref, acc_base, iota):
    for r in range(L):
        acc_ref[acc_base + r, pl.ds(0, L)] = _tree_sum(
            rows_ref[row_base + r, _ds16(j)] * q_vecs[j] for j in range(NJ))
    return _transpose_reduce(acc_ref, acc_base, iota)


def _sc_body(embs, heads, rels, tails, hn, tn, wrel,
             pos_out, neg_out,
             h_idx, r_idx, t_idx, qh, qt, pos_v,
             hn_idx, tn_idx, na0, nb0,
             acc_ref, acc_ref2, out_buf,
             sa0, sb0):
    wid = lax.axis_index("s") * NC + lax.axis_index("c")
    base = wid * BW
    iota = lax.iota(jnp.int32, L)

    def ga(buf, sem, bb):
        return pltpu.make_async_copy(embs.at[hn_idx.at[bb]], buf, sem)

    def gb(buf, sem, bb):
        return pltpu.make_async_copy(embs.at[tn_idx.at[bb]], buf, sem)

    @pl.loop(0, NCHUNK)
    def _chunk(c):
        cb = base + c * CB
        idx_copies = [
            pltpu.make_async_copy(hn.at[pl.ds(cb, CB)], hn_idx, sa0),
            pltpu.make_async_copy(tn.at[pl.ds(cb, CB)], tn_idx, sa0),
            pltpu.make_async_copy(heads.at[pl.ds(cb, CB)], h_idx, sa0),
            pltpu.make_async_copy(rels.at[pl.ds(cb, CB)], r_idx, sa0),
            pltpu.make_async_copy(tails.at[pl.ds(cb, CB)], t_idx, sa0),
        ]
        for cp_ in idx_copies:
            cp_.start()
        for cp_ in idx_copies:
            cp_.wait()

        sro_copies = [
            pltpu.make_async_copy(embs.at[h_idx], na0.at[pl.ds(0, CB)], sa0),
            pltpu.make_async_copy(wrel.at[r_idx], na0.at[pl.ds(CB, CB)], sa0),
            pltpu.make_async_copy(embs.at[t_idx], nb0.at[pl.ds(0, CB)], sb0),
        ]
        for cp_ in sro_copies:
            cp_.start()
        for cp_ in sro_copies:
            cp_.wait()

        @pl.loop(0, CB // L)
        def _q(g):
            gbase = g * L
            for r in range(L):
                acc = None
                for j in range(NJ):
                    sv = na0[gbase + r, _ds16(j)]
                    rv = na0[CB + gbase + r, _ds16(j)]
                    ov = nb0[gbase + r, _ds16(j)]
                    qhv = rv * ov
                    qtv = sv * rv
                    qh[gbase + r, _ds16(j)] = qhv
                    qt[gbase + r, _ds16(j)] = qtv
                    p = sv * qhv
                    acc = p if acc is None else acc + p
                acc_ref[r, pl.ds(0, L)] = acc
            pos_v[pl.ds(gbase, L)] = _transpose_reduce(acc_ref, 0, iota)

        pltpu.sync_copy(pos_v, pos_out.at[pl.ds(cb, CB)])
        ga(na0, sa0, 0).start()

        @pl.loop(0, CB)
        def _b(bb):
            gb(nb0, sb0, bb).start()
            qhv = [qh[bb, _ds16(j)] for j in range(NJ)]
            qtv = [qt[bb, _ds16(j)] for j in range(NJ)]

            ga(na0, sa0, bb).wait()

            @plsc.parallel_loop(0, K // L, unroll=2)
            def _g1(g):
                rb = g * L
                out_buf[bb, pl.ds(rb, L)] = _score_block(
                    na0, rb, qhv, acc_ref, rb, iota)

            @pl.when(bb < CB - 1)
            def _():
                ga(na0, sa0, bb + 1).start()

            gb(nb0, sb0, bb).wait()

            @plsc.parallel_loop(0, K // L, unroll=2)
            def _g2(g):
                rb = g * L
                out_buf[bb, pl.ds(K + rb, L)] = _score_block(
                    nb0, rb, qtv, acc_ref2, rb, iota)

        pltpu.sync_copy(out_buf, neg_out.at[pl.ds(cb, CB)])


@jax.jit
def _link_pred_sc(embs, heads, rels, tails, hn, tn, wrel):
    mesh = plsc.VectorSubcoreMesh(core_axis_name="c", subcore_axis_name="s")
    cp = pltpu.CompilerParams()
    if "needs_layout_passes" in pltpu.CompilerParams.__dataclass_fields__:
        cp = dataclasses.replace(cp, needs_layout_passes=False)
    f = pl.kernel(
        _sc_body,
        out_type=(
            jax.ShapeDtypeStruct((B,), jnp.float32),
            jax.ShapeDtypeStruct((B, 2 * K), jnp.float32),
        ),
        mesh=mesh,
        scratch_types=[
            pltpu.VMEM((CB,), jnp.int32),
            pltpu.VMEM((CB,), jnp.int32),
            pltpu.VMEM((CB,), jnp.int32),
            pltpu.VMEM((CB, D), jnp.float32),
            pltpu.VMEM((CB, D), jnp.float32),
            pltpu.VMEM((CB,), jnp.float32),
            pltpu.VMEM((CB, K), jnp.int32),
            pltpu.VMEM((CB, K), jnp.int32),
            pltpu.VMEM((K, D), jnp.float32),
            pltpu.VMEM((K, D), jnp.float32),
            pltpu.VMEM((K, L + 1), jnp.float32),
            pltpu.VMEM((K, L + 1), jnp.float32),
            pltpu.VMEM((CB, 2 * K), jnp.float32),
            pltpu.SemaphoreType.DMA,
            pltpu.SemaphoreType.DMA,
        ],
        compiler_params=cp,
    )
    return f(embs, heads, rels, tails, hn, tn, wrel)


def kernel(embs, pos_samples, head_negative_sample, tail_negative_sample,
           w_relation):
    heads = pos_samples[:, 0]
    rels = pos_samples[:, 1]
    tails = pos_samples[:, 2]
    pos, neg = _link_pred_sc(embs, heads, rels, tails,
                             head_negative_sample, tail_negative_sample,
                             w_relation)
    return pos[:, None], neg


# speedup vs baseline: 1.0222x; 1.0015x over previous
import dataclasses

import jax
import jax.numpy as jnp
from jax import lax
from jax.experimental import pallas as pl
from jax.experimental.pallas import tpu as pltpu
from jax.experimental.pallas import tpu_sc as plsc

N_NODES = 100000
D = 128
B = 8192
K = 128
L = 16
NC, NS = 2, 16
NW = NC * NS
BW = B // NW
CB = 64
NCHUNK = BW // CB
NJ = D // L


def _ds16(j):
    return pl.ds(j * L, L)


def _tree_sum(vals):
    vals = list(vals)
    while len(vals) > 1:
        nxt = [vals[2 * i] + vals[2 * i + 1] for i in range(len(vals) // 2)]
        if len(vals) % 2:
            nxt.append(vals[-1])
        vals = nxt
    return vals[0]


def _transpose_reduce(acc_ref, acc_base, iota):
    rows = iota + acc_base
    return _tree_sum(
        plsc.load_gather(acc_ref, [rows, jnp.full((L,), j, jnp.int32)])
        for j in range(L))


def _score_block(rows_ref, row_base, q_vecs, acc_ref, acc_base, iota):
    for r in range(L):
        acc_ref[acc_base + r, pl.ds(0, L)] = _tree_sum(
            rows_ref[row_base + r, _ds16(j)] * q_vecs[j] for j in range(NJ))
    return _transpose_reduce(acc_ref, acc_base, iota)


def _sc_body(embs, heads, rels, tails, hn, tn, wrel,
             pos_out, neg_out,
             h_idx, r_idx, t_idx, qh, qt, pos_v,
             hn_idx, tn_idx, na0, nb0,
             acc_ref, acc_ref2, out_buf,
             sa0, sb0):
    wid = lax.axis_index("s") * NC + lax.axis_index("c")
    base = wid * BW
    iota = lax.iota(jnp.int32, L)

    def ga(buf, sem, bb):
        return pltpu.make_async_copy(embs.at[hn_idx.at[bb]], buf, sem)

    def gb(buf, sem, bb):
        return pltpu.make_async_copy(embs.at[tn_idx.at[bb]], buf, sem)

    def out_copy(cb_):
        return pltpu.make_async_copy(
            out_buf, neg_out.at[pl.ds(cb_, CB)], sb0)

    @pl.loop(0, NCHUNK)
    def _chunk(c):
        cb = base + c * CB
        idx_copies = [
            pltpu.make_async_copy(hn.at[pl.ds(cb, CB)], hn_idx, sa0),
            pltpu.make_async_copy(tn.at[pl.ds(cb, CB)], tn_idx, sa0),
            pltpu.make_async_copy(heads.at[pl.ds(cb, CB)], h_idx, sa0),
            pltpu.make_async_copy(rels.at[pl.ds(cb, CB)], r_idx, sa0),
            pltpu.make_async_copy(tails.at[pl.ds(cb, CB)], t_idx, sa0),
        ]
        for cp_ in idx_copies:
            cp_.start()
        for cp_ in idx_copies:
            cp_.wait()

        sro_copies = [
            pltpu.make_async_copy(embs.at[h_idx], na0.at[pl.ds(0, CB)], sa0),
            pltpu.make_async_copy(wrel.at[r_idx], na0.at[pl.ds(CB, CB)], sa0),
            pltpu.make_async_copy(embs.at[t_idx], nb0.at[pl.ds(0, CB)], sb0),
        ]
        for cp_ in sro_copies:
            cp_.start()
        for cp_ in sro_copies:
            cp_.wait()

        @pl.loop(0, CB // L)
        def _q(g):
            gbase = g * L
            for r in range(L):
                acc = None
                for j in range(NJ):
                    sv = na0[gbase + r, _ds16(j)]
                    rv = na0[CB + gbase + r, _ds16(j)]
                    ov = nb0[gbase + r, _ds16(j)]
                    qhv = rv * ov
                    qtv = sv * rv
                    qh[gbase + r, _ds16(j)] = qhv
                    qt[gbase + r, _ds16(j)] = qtv
                    p = sv * qhv
                    acc = p if acc is None else acc + p
                acc_ref[r, pl.ds(0, L)] = acc
            pos_v[pl.ds(gbase, L)] = _transpose_reduce(acc_ref, 0, iota)

        pltpu.sync_copy(pos_v, pos_out.at[pl.ds(cb, CB)])

        @pl.when(c > 0)
        def _():
            out_copy(cb - CB).wait()

        ga(na0, sa0, 0).start()

        @pl.loop(0, CB)
        def _b(bb):
            gb(nb0, sb0, bb).start()
            qhv = [qh[bb, _ds16(j)] for j in range(NJ)]
            qtv = [qt[bb, _ds16(j)] for j in range(NJ)]

            ga(na0, sa0, bb).wait()

            @plsc.parallel_loop(0, K // L, unroll=2)
            def _g1(g):
                rb = g * L
                out_buf[bb, pl.ds(rb, L)] = _score_block(
                    na0, rb, qhv, acc_ref, rb, iota)

            @pl.when(bb < CB - 1)
            def _():
                ga(na0, sa0, bb + 1).start()

            gb(nb0, sb0, bb).wait()

            @plsc.parallel_loop(0, K // L, unroll=2)
            def _g2(g):
                rb = g * L
                out_buf[bb, pl.ds(K + rb, L)] = _score_block(
                    nb0, rb, qtv, acc_ref2, rb, iota)

        out_copy(cb).start()

    out_copy(base + (NCHUNK - 1) * CB).wait()


@jax.jit
def _link_pred_sc(embs, heads, rels, tails, hn, tn, wrel):
    mesh = plsc.VectorSubcoreMesh(core_axis_name="c", subcore_axis_name="s")
    cp = pltpu.CompilerParams()
    if "needs_layout_passes" in pltpu.CompilerParams.__dataclass_fields__:
        cp = dataclasses.replace(cp, needs_layout_passes=False)
    f = pl.kernel(
        _sc_body,
        out_type=(
            jax.ShapeDtypeStruct((B,), jnp.float32),
            jax.ShapeDtypeStruct((B, 2 * K), jnp.float32),
        ),
        mesh=mesh,
        scratch_types=[
            pltpu.VMEM((CB,), jnp.int32),
            pltpu.VMEM((CB,), jnp.int32),
            pltpu.VMEM((CB,), jnp.int32),
            pltpu.VMEM((CB, D), jnp.float32),
            pltpu.VMEM((CB, D), jnp.float32),
            pltpu.VMEM((CB,), jnp.float32),
            pltpu.VMEM((CB, K), jnp.int32),
            pltpu.VMEM((CB, K), jnp.int32),
            pltpu.VMEM((K, D), jnp.float32),
            pltpu.VMEM((K, D), jnp.float32),
            pltpu.VMEM((K, L + 1), jnp.float32),
            pltpu.VMEM((K, L + 1), jnp.float32),
            pltpu.VMEM((CB, 2 * K), jnp.float32),
            pltpu.SemaphoreType.DMA,
            pltpu.SemaphoreType.DMA,
        ],
        compiler_params=cp,
    )
    return f(embs, heads, rels, tails, hn, tn, wrel)


def kernel(embs, pos_samples, head_negative_sample, tail_negative_sample,
           w_relation):
    heads = pos_samples[:, 0]
    rels = pos_samples[:, 1]
    tails = pos_samples[:, 2]
    pos, neg = _link_pred_sc(embs, heads, rels, tails,
                             head_negative_sample, tail_negative_sample,
                             w_relation)
    return pos[:, None], neg
